# baseline (speedup 1.0000x reference)
import functools

import jax
import jax.numpy as jnp
from jax import lax
from jax.experimental import pallas as pl
from jax.experimental.pallas import tpu as pltpu
from jax.experimental.pallas import tpu_sc as plsc

N_NODES = 50000
N_EDGES = 800000
EMBED = 64
HIDDEN = 128

NTILES = 16
CHUNK = 128
E_PAD = 819200
EROWS = E_PAD // CHUNK
ROWS_PER_TILE_E = EROWS // NTILES

NP = 50176
ROWS_PER_TILE_N = NP // NTILES
ZCH = 112
G = 20
BM = 256
GRID_M = NP // BM


def _fill_rows(ref, nrows, val):
    v = jnp.full((16,), val, jnp.float32)

    def body(i, carry):
        ref[i, pl.ds(0, 16)] = v
        ref[i, pl.ds(16, 16)] = v
        return carry

    lax.fori_loop(0, nrows, body, 0)


def _make_sc_segsum(num_chunks, with_cnt):
    if with_cnt:
        per_core = (
            [(0, 0, 0, ROWS_PER_TILE_E),
             (None, num_chunks, 0, ROWS_PER_TILE_E // 2)],
            [(1, 1, 0, ROWS_PER_TILE_E),
             (None, num_chunks + 1, EROWS // 2, ROWS_PER_TILE_E // 2)],
        )
        n_out = num_chunks + 2
    else:
        per_core = (
            [(i, i, 0, ROWS_PER_TILE_E) for i in range(0, num_chunks, 2)],
            [(i, i, 0, ROWS_PER_TILE_E) for i in range(1, num_chunks, 2)],
        )
        n_out = num_chunks
    n_rounds = max(len(per_core[0]), len(per_core[1]))

    out_type = [jax.ShapeDtypeStruct((NP, 32), jnp.float32)] * n_out

    mesh = plsc.VectorSubcoreMesh(core_axis_name="c", subcore_axis_name="s")

    def body(*refs):
        tables = refs[:num_chunks]
        srcr = refs[num_chunks]
        dstr = refs[num_chunks + 1]
        outs = refs[num_chunks + 2: num_chunks + 2 + n_out]
        (acc, sidx_a, sidx_b, didx_a, didx_b, r0, r1, r2, r3, ones,
         gsem, isem, ssem) = refs[num_chunks + 2 + n_out:]
        sbufs = (sidx_a, sidx_b)
        dbufs = (didx_a, didx_b)
        rows = (r0, r1, r2, r3)

        c = lax.axis_index("c")
        s = lax.axis_index("s")
        nbase = s * ROWS_PER_TILE_N

        if with_cnt:
            _fill_rows(ones, CHUNK, 1.0)

        def run_round(table_idx, out_idx, erow_lo, nrows):
            _fill_rows(r0, ZCH, 0.0)

            def zero_body(k, carry):
                pltpu.sync_copy(r0.at[pl.ds(0, ZCH)],
                                acc.at[pl.ds(nbase + k * ZCH, ZCH)])
                return carry

            lax.fori_loop(0, ROWS_PER_TILE_N // ZCH, zero_body, 0)
            plsc.subcore_barrier()

            ebase = erow_lo + s * nrows
            ngroups = nrows // G

            def load_idx_async(g, k, with_src):
                pltpu.async_copy(dstr.at[pl.ds(ebase + g * G, G)],
                                 dbufs[k], isem)
                if with_src:
                    pltpu.async_copy(srcr.at[pl.ds(ebase + g * G, G)],
                                     sbufs[k], isem)

            def wait_idx(with_src):
                pltpu.make_async_copy(dstr.at[pl.ds(0, G)],
                                      didx_a, isem).wait()
                if with_src:
                    pltpu.make_async_copy(srcr.at[pl.ds(0, G)],
                                          sidx_a, isem).wait()

            npairs = ngroups // 2

            def scat_async(buf, db, j):
                pltpu.async_copy(buf, acc.at[db.at[j]], ssem, add=True)

            def wait_s():
                pltpu.make_async_copy(r0, acc.at[didx_a.at[0]], ssem).wait()

            if table_idx is None:
                load_idx_async(0, 0, False)

                def cnt_pair(it, carry):
                    for k in (0, 1):
                        g = 2 * it + k
                        wait_idx(False)
                        pl.when(g + 1 < ngroups)(
                            lambda g=g, k=k:
                                load_idx_async(g + 1, 1 - k, False))

                        def fire(j, carry2, k=k):
                            scat_async(ones, dbufs[k], j)
                            return carry2

                        def drain(j, carry2):
                            wait_s()
                            return carry2

                        lax.fori_loop(0, G, fire, 0)
                        lax.fori_loop(0, G, drain, 0)
                    return carry

                lax.fori_loop(0, npairs, cnt_pair, 0)
            else:
                table = tables[table_idx]

                def issue(sb, j, buf):
                    pltpu.async_copy(table.at[sb.at[j]], buf, gsem)

                def wait_g():
                    pltpu.make_async_copy(
                        table.at[sidx_a.at[0]], r0, gsem).wait()

                def tail4(sb, db, nsb):
                    wait_g()
                    scat_async(rows[0], db, G - 4)
                    wait_s()
                    issue(sb, G - 2, rows[2])
                    wait_g()
                    scat_async(rows[1], db, G - 3)
                    wait_s()
                    issue(sb, G - 1, rows[3])
                    wait_g()
                    scat_async(rows[2], db, G - 2)
                    wait_s()
                    if nsb is not None:
                        wait_idx(True)
                        issue(nsb, 0, rows[0])
                    wait_g()
                    scat_async(rows[3], db, G - 1)
                    wait_s()
                    if nsb is not None:
                        issue(nsb, 1, rows[1])

                def quads(sb, db, j0, nq):
                    def quad(q, carry):
                        jq = j0 + 4 * q
                        for t in range(4):
                            wait_g()
                            scat_async(rows[t], db, jq + t)
                            wait_s()
                            issue(sb, jq + t + 2, rows[(t + 2) % 4])
                        return carry

                    lax.fori_loop(0, nq, quad, 0)

                load_idx_async(0, 0, True)
                wait_idx(True)
                issue(sidx_a, 0, rows[0])
                issue(sidx_a, 1, rows[1])
                load_idx_async(1, 1, True)
                wait_g()
                scat_async(rows[0], didx_a, 0)
                issue(sidx_a, 2, rows[2])
                wait_g()
                scat_async(rows[1], didx_a, 1)
                issue(sidx_a, 3, rows[3])
                wait_g()
                scat_async(rows[2], didx_a, 2)
                wait_s()
                issue(sidx_a, 4, rows[0])
                wait_g()
                scat_async(rows[3], didx_a, 3)
                wait_s()
                issue(sidx_a, 5, rows[1])
                quads(sidx_a, didx_a, 4, (G - 8) // 4)
                tail4(sidx_a, didx_a, sidx_b)

                def mid_pair(it, carry):
                    for k in (1, 0):
                        g = 2 * it + 1 + (1 - k)
                        sb, db = sbufs[k], dbufs[k]
                        load_idx_async(g + 1, 1 - k, True)
                        quads(sb, db, 0, (G - 4) // 4)
                        tail4(sb, db, sbufs[1 - k])
                    return carry

                lax.fori_loop(0, (ngroups - 2) // 2, mid_pair, 0)

                quads(sidx_b, didx_b, 0, (G - 4) // 4)
                tail4(sidx_b, didx_b, None)
                wait_s()
                wait_s()

            plsc.subcore_barrier()

            pltpu.sync_copy(
                acc.at[pl.ds(nbase, ROWS_PER_TILE_N)],
                outs[out_idx].at[pl.ds(nbase, ROWS_PER_TILE_N)],
            )
            plsc.subcore_barrier()

        for r in range(n_rounds):
            for core_id in (0, 1):
                if r < len(per_core[core_id]):
                    item = per_core[core_id][r]
                    pl.when(c == core_id)(lambda item=item: run_round(*item))

    return pl.kernel(
        body,
        out_type=out_type,
        mesh=mesh,
        compiler_params=pltpu.CompilerParams(use_tc_tiling_on_sc=False),
        scratch_types=[
            pltpu.VMEM_SHARED((NP, 32), jnp.float32),
            pltpu.VMEM((G, CHUNK), jnp.int32),
            pltpu.VMEM((G, CHUNK), jnp.int32),
            pltpu.VMEM((G, CHUNK), jnp.int32),
            pltpu.VMEM((G, CHUNK), jnp.int32),
            pltpu.VMEM((CHUNK, 32), jnp.float32),
            pltpu.VMEM((CHUNK, 32), jnp.float32),
            pltpu.VMEM((CHUNK, 32), jnp.float32),
            pltpu.VMEM((CHUNK, 32), jnp.float32),
            pltpu.VMEM((CHUNK, 32), jnp.float32),
            pltpu.SemaphoreType.DMA,
            pltpu.SemaphoreType.DMA,
            pltpu.SemaphoreType.DMA,
        ],
    )


_sc_segsum_l1 = _make_sc_segsum(2, with_cnt=True)
_sc_segsum_l2 = _make_sc_segsum(4, with_cnt=False)


def _lin1_body(x, w, b, out):
    out[...] = (
        jnp.dot(x[...], w[...], preferred_element_type=jnp.float32) + b[...]
    )


def _lin4_body(h0, h1, h2, h3, w, b, out):
    hh = jnp.concatenate([h0[...], h1[...], h2[...], h3[...]], axis=1)
    out[...] = (
        jnp.dot(hh, w[...], preferred_element_type=jnp.float32) + b[...]
    )


def _tc1_body(a0, a1, cnta, cntb, xr, w1l, h0, h1, h2, h3):
    r = 1.0 / jnp.maximum(cnta[:, :1] + cntb[:, :1], 1.0)
    m = jnp.concatenate([a0[...], a1[...]], axis=1) * r
    h = jnp.maximum(
        jnp.dot(m, w1l[...], preferred_element_type=jnp.float32) + xr[...],
        0.0)
    h0[...] = h[:, 0:32]
    h1[...] = h[:, 32:64]
    h2[...] = h[:, 64:96]
    h3[...] = h[:, 96:128]


def _tc2_body(a0, a1, a2, a3, cnta, cntb, hr, w2l, wfc, bfc, out):
    r = 1.0 / jnp.maximum(cnta[:, :1] + cntb[:, :1], 1.0)
    m = jnp.concatenate([a0[...], a1[...], a2[...], a3[...]], axis=1) * r
    hrelu = jnp.maximum(
        jnp.dot(m, w2l[...], preferred_element_type=jnp.float32) + hr[...],
        0.0)
    out[...] = (
        jnp.dot(hrelu, wfc[...], preferred_element_type=jnp.float32) + bfc[...]
    )


def _row_spec(cols):
    return pl.BlockSpec((BM, cols), lambda i: (i, 0))


def _full_spec(shape):
    nd = len(shape)
    return pl.BlockSpec(shape, lambda i: (0,) * nd)


_lin1 = pl.pallas_call(
    _lin1_body,
    grid=(GRID_M,),
    in_specs=[
        _row_spec(EMBED),
        _full_spec((EMBED, HIDDEN)), _full_spec((1, HIDDEN)),
    ],
    out_specs=_row_spec(HIDDEN),
    out_shape=jax.ShapeDtypeStruct((NP, HIDDEN), jnp.float32),
)

_lin4 = pl.pallas_call(
    _lin4_body,
    grid=(GRID_M,),
    in_specs=[
        _row_spec(32), _row_spec(32), _row_spec(32), _row_spec(32),
        _full_spec((HIDDEN, HIDDEN)), _full_spec((1, HIDDEN)),
    ],
    out_specs=_row_spec(HIDDEN),
    out_shape=jax.ShapeDtypeStruct((NP, HIDDEN), jnp.float32),
)

_tc1 = pl.pallas_call(
    _tc1_body,
    grid=(GRID_M,),
    in_specs=[
        _row_spec(32), _row_spec(32), _row_spec(32), _row_spec(32),
        _row_spec(HIDDEN),
        _full_spec((EMBED, HIDDEN)),
    ],
    out_specs=[_row_spec(32)] * 4,
    out_shape=[jax.ShapeDtypeStruct((NP, 32), jnp.float32)] * 4,
)

_tc2 = pl.pallas_call(
    _tc2_body,
    grid=(GRID_M,),
    in_specs=[
        _row_spec(32), _row_spec(32), _row_spec(32), _row_spec(32),
        _row_spec(32), _row_spec(32),
        _row_spec(HIDDEN),
        _full_spec((HIDDEN, HIDDEN)),
        _full_spec((HIDDEN, EMBED)), _full_spec((1, EMBED)),
    ],
    out_specs=_row_spec(EMBED),
    out_shape=jax.ShapeDtypeStruct((NP, EMBED), jnp.float32),
)


@jax.jit
def kernel(x, edge_index, W1l, W1r, b1, W2l, W2r, b2, Wfc, bfc):
    ei = edge_index.astype(jnp.int32)
    pad = E_PAD - N_EDGES
    srcp = jnp.concatenate([ei[0], jnp.zeros((pad,), jnp.int32)])
    dstp = jnp.concatenate([ei[1], jnp.full((pad,), N_NODES, jnp.int32)])
    srcp = srcp.reshape(EROWS, CHUNK)
    dstp = dstp.reshape(EROWS, CHUNK)

    xp = jnp.pad(x, ((0, NP - N_NODES), (0, 0)))
    x0 = xp[:, 0:32]
    x1 = xp[:, 32:64]

    a0, a1, cnta, cntb = _sc_segsum_l1(x0, x1, srcp, dstp)
    xr = _lin1(xp, W1r, b1.reshape(1, HIDDEN))
    h0, h1, h2, h3 = _tc1(a0, a1, cnta, cntb, xr, W1l)
    g0, g1, g2, g3 = _sc_segsum_l2(h0, h1, h2, h3, srcp, dstp)
    hr = _lin4(h0, h1, h2, h3, W2r, b2.reshape(1, HIDDEN))
    out = _tc2(g0, g1, g2, g3, cnta, cntb, hr,
               W2l, Wfc, bfc.reshape(1, EMBED))
    return out[:N_NODES]


# speedup vs baseline: 1.0578x; 1.0578x over previous
import functools

import jax
import jax.numpy as jnp
from jax import lax
from jax.experimental import pallas as pl
from jax.experimental.pallas import tpu as pltpu
from jax.experimental.pallas import tpu_sc as plsc

N_NODES = 50000
N_EDGES = 800000
EMBED = 64
HIDDEN = 128

NTILES = 16
CHUNK = 128
E_PAD = 819200
EROWS = E_PAD // CHUNK
ROWS_PER_TILE_E = EROWS // NTILES

NP = 50176
ROWS_PER_TILE_N = NP // NTILES
ZCH = 112
G = 20
BM = 256
GRID_M = NP // BM


def _fill_rows(ref, nrows, val):
    v = jnp.full((16,), val, jnp.float32)

    def body(i, carry):
        ref[i, pl.ds(0, 16)] = v
        ref[i, pl.ds(16, 16)] = v
        return carry

    lax.fori_loop(0, nrows, body, 0)


def _make_sc_segsum(num_chunks, with_cnt):
    per_core = (
        [(i, i, with_cnt and i == 0) for i in range(0, num_chunks, 2)],
        [(i, i, False) for i in range(1, num_chunks, 2)],
    )
    n_rounds = max(len(per_core[0]), len(per_core[1]))

    n_out = num_chunks + (1 if with_cnt else 0)
    out_type = [jax.ShapeDtypeStruct((NP, 32), jnp.float32)] * num_chunks
    if with_cnt:
        out_type = out_type + [jax.ShapeDtypeStruct((NP,), jnp.float32)]

    mesh = plsc.VectorSubcoreMesh(core_axis_name="c", subcore_axis_name="s")

    def body(*refs):
        tables = refs[:num_chunks]
        srcr = refs[num_chunks]
        dstr = refs[num_chunks + 1]
        outs = refs[num_chunks + 2: num_chunks + 2 + n_out]
        rest = refs[num_chunks + 2 + n_out:]
        if with_cnt:
            (acc, cnt_acc, sidx_a, sidx_b, didx_a, didx_b, r0, r1, r2, r3,
             zs, os, gsem, isem, ssem, csem) = rest
        else:
            (acc, sidx_a, sidx_b, didx_a, didx_b, r0, r1, r2, r3,
             gsem, isem, ssem) = rest
        sbufs = (sidx_a, sidx_b)
        dbufs = (didx_a, didx_b)
        rows = (r0, r1, r2, r3)

        c = lax.axis_index("c")
        s = lax.axis_index("s")
        nbase = s * ROWS_PER_TILE_N

        if with_cnt:
            zv = jnp.zeros((16,), jnp.float32)
            ov = jnp.ones((16,), jnp.float32)
            for i in range(CHUNK // 16):
                zs[pl.ds(16 * i, 16)] = zv
                os[pl.ds(16 * i, 16)] = ov

        def run_round(table_idx, out_idx, do_cnt):
            erow_lo, nrows = 0, ROWS_PER_TILE_E
            _fill_rows(r0, ZCH, 0.0)

            def zero_body(k, carry):
                pltpu.sync_copy(r0.at[pl.ds(0, ZCH)],
                                acc.at[pl.ds(nbase + k * ZCH, ZCH)])
                return carry

            lax.fori_loop(0, ROWS_PER_TILE_N // ZCH, zero_body, 0)
            if do_cnt:
                def zero_cnt(k, carry):
                    pltpu.sync_copy(
                        zs.at[pl.ds(0, ZCH)],
                        cnt_acc.at[pl.ds(nbase + k * ZCH, ZCH)])
                    return carry

                lax.fori_loop(0, ROWS_PER_TILE_N // ZCH, zero_cnt, 0)
            plsc.subcore_barrier()

            ebase = erow_lo + s * nrows
            ngroups = nrows // G

            def load_idx_async(g, k, with_src):
                pltpu.async_copy(dstr.at[pl.ds(ebase + g * G, G)],
                                 dbufs[k], isem)
                if with_src:
                    pltpu.async_copy(srcr.at[pl.ds(ebase + g * G, G)],
                                     sbufs[k], isem)

            def wait_idx(with_src):
                pltpu.make_async_copy(dstr.at[pl.ds(0, G)],
                                      didx_a, isem).wait()
                if with_src:
                    pltpu.make_async_copy(srcr.at[pl.ds(0, G)],
                                          sidx_a, isem).wait()

            def scat_async(buf, db, j):
                pltpu.async_copy(buf, acc.at[db.at[j]], ssem, add=True)
                if do_cnt:
                    pltpu.async_copy(os, cnt_acc.at[db.at[j]], csem,
                                     add=True)

            def wait_s():
                pltpu.make_async_copy(r0, acc.at[didx_a.at[0]], ssem).wait()

            if True:
                table = tables[table_idx]

                def issue(sb, j, buf):
                    pltpu.async_copy(table.at[sb.at[j]], buf, gsem)

                def wait_g():
                    pltpu.make_async_copy(
                        table.at[sidx_a.at[0]], r0, gsem).wait()

                def tail4(sb, db, nsb):
                    wait_g()
                    scat_async(rows[0], db, G - 4)
                    wait_s()
                    issue(sb, G - 2, rows[2])
                    wait_g()
                    scat_async(rows[1], db, G - 3)
                    wait_s()
                    issue(sb, G - 1, rows[3])
                    wait_g()
                    scat_async(rows[2], db, G - 2)
                    wait_s()
                    if nsb is not None:
                        wait_idx(True)
                        issue(nsb, 0, rows[0])
                    wait_g()
                    scat_async(rows[3], db, G - 1)
                    wait_s()
                    if nsb is not None:
                        issue(nsb, 1, rows[1])

                def quads(sb, db, j0, nq):
                    def quad(q, carry):
                        jq = j0 + 4 * q
                        for t in range(4):
                            wait_g()
                            scat_async(rows[t], db, jq + t)
                            wait_s()
                            issue(sb, jq + t + 2, rows[(t + 2) % 4])
                        return carry

                    lax.fori_loop(0, nq, quad, 0)

                load_idx_async(0, 0, True)
                wait_idx(True)
                issue(sidx_a, 0, rows[0])
                issue(sidx_a, 1, rows[1])
                load_idx_async(1, 1, True)
                wait_g()
                scat_async(rows[0], didx_a, 0)
                issue(sidx_a, 2, rows[2])
                wait_g()
                scat_async(rows[1], didx_a, 1)
                issue(sidx_a, 3, rows[3])
                wait_g()
                scat_async(rows[2], didx_a, 2)
                wait_s()
                issue(sidx_a, 4, rows[0])
                wait_g()
                scat_async(rows[3], didx_a, 3)
                wait_s()
                issue(sidx_a, 5, rows[1])
                quads(sidx_a, didx_a, 4, (G - 8) // 4)
                tail4(sidx_a, didx_a, sidx_b)

                def mid_pair(it, carry):
                    for k in (1, 0):
                        g = 2 * it + 1 + (1 - k)
                        sb, db = sbufs[k], dbufs[k]
                        load_idx_async(g + 1, 1 - k, True)
                        quads(sb, db, 0, (G - 4) // 4)
                        tail4(sb, db, sbufs[1 - k])
                    return carry

                lax.fori_loop(0, (ngroups - 2) // 2, mid_pair, 0)

                quads(sidx_b, didx_b, 0, (G - 4) // 4)
                tail4(sidx_b, didx_b, None)
                wait_s()
                wait_s()

                if do_cnt:
                    def drain_cnt(j, carry):
                        pltpu.make_async_copy(
                            os, cnt_acc.at[didx_a.at[0]], csem).wait()
                        return carry

                    lax.fori_loop(0, nrows, drain_cnt, 0)

            plsc.subcore_barrier()

            pltpu.sync_copy(
                acc.at[pl.ds(nbase, ROWS_PER_TILE_N)],
                outs[out_idx].at[pl.ds(nbase, ROWS_PER_TILE_N)],
            )
            if do_cnt:
                pltpu.sync_copy(
                    cnt_acc.at[pl.ds(nbase, ROWS_PER_TILE_N)],
                    outs[num_chunks].at[pl.ds(nbase, ROWS_PER_TILE_N)],
                )
            plsc.subcore_barrier()

        for r in range(n_rounds):
            for core_id in (0, 1):
                if r < len(per_core[core_id]):
                    item = per_core[core_id][r]
                    pl.when(c == core_id)(lambda item=item: run_round(*item))

    scratch = [
        pltpu.VMEM_SHARED((NP, 32), jnp.float32),
    ]
    if with_cnt:
        scratch.append(pltpu.VMEM_SHARED((NP,), jnp.float32))
    scratch += [
        pltpu.VMEM((G, CHUNK), jnp.int32),
        pltpu.VMEM((G, CHUNK), jnp.int32),
        pltpu.VMEM((G, CHUNK), jnp.int32),
        pltpu.VMEM((G, CHUNK), jnp.int32),
        *([pltpu.VMEM((CHUNK, 32), jnp.float32)] * 4),
    ]
    if with_cnt:
        scratch += [
            pltpu.VMEM((CHUNK,), jnp.float32),
            pltpu.VMEM((CHUNK,), jnp.float32),
        ]
    scratch += [
        pltpu.SemaphoreType.DMA,
        pltpu.SemaphoreType.DMA,
        pltpu.SemaphoreType.DMA,
    ]
    if with_cnt:
        scratch.append(pltpu.SemaphoreType.DMA)

    return pl.kernel(
        body,
        out_type=out_type,
        mesh=mesh,
        compiler_params=pltpu.CompilerParams(use_tc_tiling_on_sc=False),
        scratch_types=scratch,
    )


_sc_segsum_l1 = _make_sc_segsum(2, with_cnt=True)
_sc_segsum_l2 = _make_sc_segsum(4, with_cnt=False)


def _lin1_body(x, w, b, out):
    out[...] = (
        jnp.dot(x[...], w[...], preferred_element_type=jnp.float32) + b[...]
    )


def _lin4_body(h0, h1, h2, h3, w, b, out):
    hh = jnp.concatenate([h0[...], h1[...], h2[...], h3[...]], axis=1)
    out[...] = (
        jnp.dot(hh, w[...], preferred_element_type=jnp.float32) + b[...]
    )


def _tc1_body(a0, a1, cnt, xr, w1l, h0, h1, h2, h3):
    r = 1.0 / jnp.maximum(cnt[...], 1.0)
    m = jnp.concatenate([a0[...], a1[...]], axis=1) * r
    h = jnp.maximum(
        jnp.dot(m, w1l[...], preferred_element_type=jnp.float32) + xr[...],
        0.0)
    h0[...] = h[:, 0:32]
    h1[...] = h[:, 32:64]
    h2[...] = h[:, 64:96]
    h3[...] = h[:, 96:128]


def _tc2_body(a0, a1, a2, a3, cnt, hr, w2l, wfc, bfc, out):
    r = 1.0 / jnp.maximum(cnt[...], 1.0)
    m = jnp.concatenate([a0[...], a1[...], a2[...], a3[...]], axis=1) * r
    hrelu = jnp.maximum(
        jnp.dot(m, w2l[...], preferred_element_type=jnp.float32) + hr[...],
        0.0)
    out[...] = (
        jnp.dot(hrelu, wfc[...], preferred_element_type=jnp.float32) + bfc[...]
    )


def _row_spec(cols):
    return pl.BlockSpec((BM, cols), lambda i: (i, 0))


def _full_spec(shape):
    nd = len(shape)
    return pl.BlockSpec(shape, lambda i: (0,) * nd)


_lin1 = pl.pallas_call(
    _lin1_body,
    grid=(GRID_M,),
    in_specs=[
        _row_spec(EMBED),
        _full_spec((EMBED, HIDDEN)), _full_spec((1, HIDDEN)),
    ],
    out_specs=_row_spec(HIDDEN),
    out_shape=jax.ShapeDtypeStruct((NP, HIDDEN), jnp.float32),
)

_lin4 = pl.pallas_call(
    _lin4_body,
    grid=(GRID_M,),
    in_specs=[
        _row_spec(32), _row_spec(32), _row_spec(32), _row_spec(32),
        _full_spec((HIDDEN, HIDDEN)), _full_spec((1, HIDDEN)),
    ],
    out_specs=_row_spec(HIDDEN),
    out_shape=jax.ShapeDtypeStruct((NP, HIDDEN), jnp.float32),
)

_tc1 = pl.pallas_call(
    _tc1_body,
    grid=(GRID_M,),
    in_specs=[
        _row_spec(32), _row_spec(32), _row_spec(1),
        _row_spec(HIDDEN),
        _full_spec((EMBED, HIDDEN)),
    ],
    out_specs=[_row_spec(32)] * 4,
    out_shape=[jax.ShapeDtypeStruct((NP, 32), jnp.float32)] * 4,
)

_tc2 = pl.pallas_call(
    _tc2_body,
    grid=(GRID_M,),
    in_specs=[
        _row_spec(32), _row_spec(32), _row_spec(32), _row_spec(32),
        _row_spec(1),
        _row_spec(HIDDEN),
        _full_spec((HIDDEN, HIDDEN)),
        _full_spec((HIDDEN, EMBED)), _full_spec((1, EMBED)),
    ],
    out_specs=_row_spec(EMBED),
    out_shape=jax.ShapeDtypeStruct((NP, EMBED), jnp.float32),
)


@jax.jit
def kernel(x, edge_index, W1l, W1r, b1, W2l, W2r, b2, Wfc, bfc):
    ei = edge_index.astype(jnp.int32)
    pad = E_PAD - N_EDGES
    srcp = jnp.concatenate([ei[0], jnp.zeros((pad,), jnp.int32)])
    dstp = jnp.concatenate([ei[1], jnp.full((pad,), N_NODES, jnp.int32)])
    srcp = srcp.reshape(EROWS, CHUNK)
    dstp = dstp.reshape(EROWS, CHUNK)

    xp = jnp.pad(x, ((0, NP - N_NODES), (0, 0)))
    x0 = xp[:, 0:32]
    x1 = xp[:, 32:64]

    a0, a1, cnt = _sc_segsum_l1(x0, x1, srcp, dstp)
    cnt = cnt.reshape(NP, 1)
    xr = _lin1(xp, W1r, b1.reshape(1, HIDDEN))
    h0, h1, h2, h3 = _tc1(a0, a1, cnt, xr, W1l)
    g0, g1, g2, g3 = _sc_segsum_l2(h0, h1, h2, h3, srcp, dstp)
    hr = _lin4(h0, h1, h2, h3, W2r, b2.reshape(1, HIDDEN))
    out = _tc2(g0, g1, g2, g3, cnt, hr,
               W2l, Wfc, bfc.reshape(1, EMBED))
    return out[:N_NODES]


# speedup vs baseline: 1.1663x; 1.1026x over previous
import functools

import jax
import jax.numpy as jnp
from jax import lax
from jax.experimental import pallas as pl
from jax.experimental.pallas import tpu as pltpu
from jax.experimental.pallas import tpu_sc as plsc

N_NODES = 50000
N_EDGES = 800000
EMBED = 64
HIDDEN = 128

NTILES = 16
CHUNK = 128
E_PAD = 819200
EROWS = E_PAD // CHUNK
ROWS_PER_TILE_E = EROWS // NTILES

NP = 50176
ROWS_PER_TILE_N = NP // NTILES
ZCH = 112
G = 20
BM = 1024
GRID_M = NP // BM


def _fill_rows(ref, nrows, val):
    v = jnp.full((16,), val, jnp.float32)

    def body(i, carry):
        ref[i, pl.ds(0, 16)] = v
        ref[i, pl.ds(16, 16)] = v
        return carry

    lax.fori_loop(0, nrows, body, 0)


def _make_sc_segsum(num_chunks, with_cnt):
    per_core = (
        [(i, i, with_cnt and i == 0) for i in range(0, num_chunks, 2)],
        [(i, i, False) for i in range(1, num_chunks, 2)],
    )
    n_rounds = max(len(per_core[0]), len(per_core[1]))

    n_out = num_chunks + (1 if with_cnt else 0)
    out_type = [jax.ShapeDtypeStruct((NP, 32), jnp.float32)] * num_chunks
    if with_cnt:
        out_type = out_type + [jax.ShapeDtypeStruct((NP,), jnp.float32)]

    mesh = plsc.VectorSubcoreMesh(core_axis_name="c", subcore_axis_name="s")

    def body(*refs):
        tables = refs[:num_chunks]
        srcr = refs[num_chunks]
        dstr = refs[num_chunks + 1]
        outs = refs[num_chunks + 2: num_chunks + 2 + n_out]
        rest = refs[num_chunks + 2 + n_out:]
        if with_cnt:
            (acc, cnt_acc, sidx_a, sidx_b, didx_a, didx_b, r0, r1, r2, r3,
             zs, os, gsem, isem, ssem, csem) = rest
        else:
            (acc, sidx_a, sidx_b, didx_a, didx_b, r0, r1, r2, r3,
             gsem, isem, ssem) = rest
        sbufs = (sidx_a, sidx_b)
        dbufs = (didx_a, didx_b)
        rows = (r0, r1, r2, r3)

        c = lax.axis_index("c")
        s = lax.axis_index("s")
        nbase = s * ROWS_PER_TILE_N

        if with_cnt:
            zv = jnp.zeros((16,), jnp.float32)
            ov = jnp.ones((16,), jnp.float32)
            for i in range(CHUNK // 16):
                zs[pl.ds(16 * i, 16)] = zv
                os[pl.ds(16 * i, 16)] = ov

        def run_round(table_idx, out_idx, do_cnt):
            erow_lo, nrows = 0, ROWS_PER_TILE_E
            _fill_rows(r0, ZCH, 0.0)

            def zero_body(k, carry):
                pltpu.sync_copy(r0.at[pl.ds(0, ZCH)],
                                acc.at[pl.ds(nbase + k * ZCH, ZCH)])
                return carry

            lax.fori_loop(0, ROWS_PER_TILE_N // ZCH, zero_body, 0)
            if do_cnt:
                def zero_cnt(k, carry):
                    pltpu.sync_copy(
                        zs.at[pl.ds(0, ZCH)],
                        cnt_acc.at[pl.ds(nbase + k * ZCH, ZCH)])
                    return carry

                lax.fori_loop(0, ROWS_PER_TILE_N // ZCH, zero_cnt, 0)
            plsc.subcore_barrier()

            ebase = erow_lo + s * nrows
            ngroups = nrows // G

            def load_idx_async(g, k, with_src):
                pltpu.async_copy(dstr.at[pl.ds(ebase + g * G, G)],
                                 dbufs[k], isem)
                if with_src:
                    pltpu.async_copy(srcr.at[pl.ds(ebase + g * G, G)],
                                     sbufs[k], isem)

            def wait_idx(with_src):
                pltpu.make_async_copy(dstr.at[pl.ds(0, G)],
                                      didx_a, isem).wait()
                if with_src:
                    pltpu.make_async_copy(srcr.at[pl.ds(0, G)],
                                          sidx_a, isem).wait()

            def scat_async(buf, db, j):
                pltpu.async_copy(buf, acc.at[db.at[j]], ssem, add=True)
                if do_cnt:
                    pltpu.async_copy(os, cnt_acc.at[db.at[j]], csem,
                                     add=True)

            def wait_s():
                pltpu.make_async_copy(r0, acc.at[didx_a.at[0]], ssem).wait()

            if True:
                table = tables[table_idx]

                def issue(sb, j, buf):
                    pltpu.async_copy(table.at[sb.at[j]], buf, gsem)

                def wait_g():
                    pltpu.make_async_copy(
                        table.at[sidx_a.at[0]], r0, gsem).wait()

                def tail4(sb, db, nsb):
                    wait_g()
                    scat_async(rows[0], db, G - 4)
                    wait_s()
                    issue(sb, G - 2, rows[2])
                    wait_g()
                    scat_async(rows[1], db, G - 3)
                    wait_s()
                    issue(sb, G - 1, rows[3])
                    wait_g()
                    scat_async(rows[2], db, G - 2)
                    wait_s()
                    if nsb is not None:
                        wait_idx(True)
                        issue(nsb, 0, rows[0])
                    wait_g()
                    scat_async(rows[3], db, G - 1)
                    wait_s()
                    if nsb is not None:
                        issue(nsb, 1, rows[1])

                def quads(sb, db, j0, nq):
                    def quad(q, carry):
                        jq = j0 + 4 * q
                        for t in range(4):
                            wait_g()
                            scat_async(rows[t], db, jq + t)
                            wait_s()
                            issue(sb, jq + t + 2, rows[(t + 2) % 4])
                        return carry

                    lax.fori_loop(0, nq, quad, 0)

                load_idx_async(0, 0, True)
                wait_idx(True)
                issue(sidx_a, 0, rows[0])
                issue(sidx_a, 1, rows[1])
                load_idx_async(1, 1, True)
                wait_g()
                scat_async(rows[0], didx_a, 0)
                issue(sidx_a, 2, rows[2])
                wait_g()
                scat_async(rows[1], didx_a, 1)
                issue(sidx_a, 3, rows[3])
                wait_g()
                scat_async(rows[2], didx_a, 2)
                wait_s()
                issue(sidx_a, 4, rows[0])
                wait_g()
                scat_async(rows[3], didx_a, 3)
                wait_s()
                issue(sidx_a, 5, rows[1])
                quads(sidx_a, didx_a, 4, (G - 8) // 4)
                tail4(sidx_a, didx_a, sidx_b)

                def mid_pair(it, carry):
                    for k in (1, 0):
                        g = 2 * it + 1 + (1 - k)
                        sb, db = sbufs[k], dbufs[k]
                        load_idx_async(g + 1, 1 - k, True)
                        quads(sb, db, 0, (G - 4) // 4)
                        tail4(sb, db, sbufs[1 - k])
                    return carry

                lax.fori_loop(0, (ngroups - 2) // 2, mid_pair, 0)

                quads(sidx_b, didx_b, 0, (G - 4) // 4)
                tail4(sidx_b, didx_b, None)
                wait_s()
                wait_s()

                if do_cnt:
                    def drain_cnt(j, carry):
                        pltpu.make_async_copy(
                            os, cnt_acc.at[didx_a.at[0]], csem).wait()
                        return carry

                    lax.fori_loop(0, nrows, drain_cnt, 0)

            plsc.subcore_barrier()

            pltpu.sync_copy(
                acc.at[pl.ds(nbase, ROWS_PER_TILE_N)],
                outs[out_idx].at[pl.ds(nbase, ROWS_PER_TILE_N)],
            )
            if do_cnt:
                pltpu.sync_copy(
                    cnt_acc.at[pl.ds(nbase, ROWS_PER_TILE_N)],
                    outs[num_chunks].at[pl.ds(nbase, ROWS_PER_TILE_N)],
                )
            plsc.subcore_barrier()

        for r in range(n_rounds):
            for core_id in (0, 1):
                if r < len(per_core[core_id]):
                    item = per_core[core_id][r]
                    pl.when(c == core_id)(lambda item=item: run_round(*item))

    scratch = [
        pltpu.VMEM_SHARED((NP, 32), jnp.float32),
    ]
    if with_cnt:
        scratch.append(pltpu.VMEM_SHARED((NP,), jnp.float32))
    scratch += [
        pltpu.VMEM((G, CHUNK), jnp.int32),
        pltpu.VMEM((G, CHUNK), jnp.int32),
        pltpu.VMEM((G, CHUNK), jnp.int32),
        pltpu.VMEM((G, CHUNK), jnp.int32),
        *([pltpu.VMEM((CHUNK, 32), jnp.float32)] * 4),
    ]
    if with_cnt:
        scratch += [
            pltpu.VMEM((CHUNK,), jnp.float32),
            pltpu.VMEM((CHUNK,), jnp.float32),
        ]
    scratch += [
        pltpu.SemaphoreType.DMA,
        pltpu.SemaphoreType.DMA,
        pltpu.SemaphoreType.DMA,
    ]
    if with_cnt:
        scratch.append(pltpu.SemaphoreType.DMA)

    return pl.kernel(
        body,
        out_type=out_type,
        mesh=mesh,
        compiler_params=pltpu.CompilerParams(use_tc_tiling_on_sc=False),
        scratch_types=scratch,
    )


_sc_segsum_l1 = _make_sc_segsum(2, with_cnt=True)
_sc_segsum_l2 = _make_sc_segsum(4, with_cnt=False)


def _lin1_body(x, w, b, out):
    out[...] = (
        jnp.dot(x[...], w[...], preferred_element_type=jnp.float32) + b[...]
    )


def _lin4_body(h0, h1, h2, h3, w, b, out):
    hh = jnp.concatenate([h0[...], h1[...], h2[...], h3[...]], axis=1)
    out[...] = (
        jnp.dot(hh, w[...], preferred_element_type=jnp.float32) + b[...]
    )


def _tc1_body(a0, a1, cnt, xr, w1l, h0, h1, h2, h3):
    r = 1.0 / jnp.maximum(cnt[...], 1.0)
    m = jnp.concatenate([a0[...], a1[...]], axis=1) * r
    h = jnp.maximum(
        jnp.dot(m, w1l[...], preferred_element_type=jnp.float32) + xr[...],
        0.0)
    h0[...] = h[:, 0:32]
    h1[...] = h[:, 32:64]
    h2[...] = h[:, 64:96]
    h3[...] = h[:, 96:128]


def _tc2_body(a0, a1, a2, a3, cnt, hr, w2l, wfc, bfc, out):
    r = 1.0 / jnp.maximum(cnt[...], 1.0)
    m = jnp.concatenate([a0[...], a1[...], a2[...], a3[...]], axis=1) * r
    hrelu = jnp.maximum(
        jnp.dot(m, w2l[...], preferred_element_type=jnp.float32) + hr[...],
        0.0)
    out[...] = (
        jnp.dot(hrelu, wfc[...], preferred_element_type=jnp.float32) + bfc[...]
    )


def _row_spec(cols):
    return pl.BlockSpec((BM, cols), lambda i: (i, 0))


def _full_spec(shape):
    nd = len(shape)
    return pl.BlockSpec(shape, lambda i: (0,) * nd)


_lin1 = pl.pallas_call(
    _lin1_body,
    grid=(GRID_M,),
    in_specs=[
        _row_spec(EMBED),
        _full_spec((EMBED, HIDDEN)), _full_spec((1, HIDDEN)),
    ],
    out_specs=_row_spec(HIDDEN),
    out_shape=jax.ShapeDtypeStruct((NP, HIDDEN), jnp.float32),
)

_lin4 = pl.pallas_call(
    _lin4_body,
    grid=(GRID_M,),
    in_specs=[
        _row_spec(32), _row_spec(32), _row_spec(32), _row_spec(32),
        _full_spec((HIDDEN, HIDDEN)), _full_spec((1, HIDDEN)),
    ],
    out_specs=_row_spec(HIDDEN),
    out_shape=jax.ShapeDtypeStruct((NP, HIDDEN), jnp.float32),
)

_tc1 = pl.pallas_call(
    _tc1_body,
    grid=(GRID_M,),
    in_specs=[
        _row_spec(32), _row_spec(32), _row_spec(1),
        _row_spec(HIDDEN),
        _full_spec((EMBED, HIDDEN)),
    ],
    out_specs=[_row_spec(32)] * 4,
    out_shape=[jax.ShapeDtypeStruct((NP, 32), jnp.float32)] * 4,
)

_tc2 = pl.pallas_call(
    _tc2_body,
    grid=(GRID_M,),
    in_specs=[
        _row_spec(32), _row_spec(32), _row_spec(32), _row_spec(32),
        _row_spec(1),
        _row_spec(HIDDEN),
        _full_spec((HIDDEN, HIDDEN)),
        _full_spec((HIDDEN, EMBED)), _full_spec((1, EMBED)),
    ],
    out_specs=_row_spec(EMBED),
    out_shape=jax.ShapeDtypeStruct((NP, EMBED), jnp.float32),
)


@jax.jit
def kernel(x, edge_index, W1l, W1r, b1, W2l, W2r, b2, Wfc, bfc):
    ei = edge_index.astype(jnp.int32)
    pad = E_PAD - N_EDGES
    srcp = jnp.concatenate([ei[0], jnp.zeros((pad,), jnp.int32)])
    dstp = jnp.concatenate([ei[1], jnp.full((pad,), N_NODES, jnp.int32)])
    srcp = srcp.reshape(EROWS, CHUNK)
    dstp = dstp.reshape(EROWS, CHUNK)

    xp = jnp.pad(x, ((0, NP - N_NODES), (0, 0)))
    x0 = xp[:, 0:32]
    x1 = xp[:, 32:64]

    a0, a1, cnt = _sc_segsum_l1(x0, x1, srcp, dstp)
    cnt = cnt.reshape(NP, 1)
    xr = _lin1(xp, W1r, b1.reshape(1, HIDDEN))
    h0, h1, h2, h3 = _tc1(a0, a1, cnt, xr, W1l)
    g0, g1, g2, g3 = _sc_segsum_l2(h0, h1, h2, h3, srcp, dstp)
    hr = _lin4(h0, h1, h2, h3, W2r, b2.reshape(1, HIDDEN))
    out = _tc2(g0, g1, g2, g3, cnt, hr,
               W2l, Wfc, bfc.reshape(1, EMBED))
    return out[:N_NODES]
